# baseline (speedup 1.0000x reference)
import functools

import jax
import jax.numpy as jnp
from jax import lax
from jax.experimental import pallas as pl
from jax.experimental.pallas import tpu as pltpu
from jax.experimental.pallas import tpu_sc as plsc

B = 16384
D = 64
L = 16
NC = 2
NS = 16
NW = NC * NS
BPW = B // NW
CHUNK = 128
NCHUNK = BPW // CHUNK
NGROUP = BPW // L

_mesh = plsc.VectorSubcoreMesh(core_axis_name="c", subcore_axis_name="s")


@functools.partial(
    pl.kernel,
    mesh=_mesh,
    compiler_params=pltpu.CompilerParams(
        needs_layout_passes=False, use_tc_tiling_on_sc=False),
    out_type=jax.ShapeDtypeStruct((B,), jnp.float32),
    scratch_types=[
        pltpu.VMEM((NCHUNK, CHUNK), jnp.int32),
        pltpu.VMEM((NCHUNK, CHUNK), jnp.int32),
        pltpu.VMEM((BPW, D), jnp.float32),
        pltpu.VMEM((BPW, D), jnp.float32),
        pltpu.VMEM((D,), jnp.float32),
        pltpu.VMEM((L,), jnp.float32),
        pltpu.VMEM((BPW,), jnp.float32),
        pltpu.SemaphoreType.DMA,
        pltpu.SemaphoreType.DMA,
    ],
)
def _sc_kernel(uid_hbm, iid_hbm, ut_hbm, it_hbm, w_hbm, b_hbm, out_hbm,
               uidx_v, iidx_v, urows_v, irows_v, w_v, b_v, out_v,
               usem, isem):
    wid = lax.axis_index("s") * NC + lax.axis_index("c")
    base = wid * BPW

    for c in range(NCHUNK):
        pltpu.sync_copy(uid_hbm.at[pl.ds(base + c * CHUNK, CHUNK)],
                        uidx_v.at[c])
        pltpu.sync_copy(iid_hbm.at[pl.ds(base + c * CHUNK, CHUNK)],
                        iidx_v.at[c])
    pltpu.sync_copy(w_hbm, w_v)
    pltpu.sync_copy(b_hbm, b_v)

    copies = []
    for c in range(NCHUNK):
        copies.append(pltpu.async_copy(
            ut_hbm.at[uidx_v.at[c]], urows_v.at[pl.ds(c * CHUNK, CHUNK)],
            usem))
        copies.append(pltpu.async_copy(
            it_hbm.at[iidx_v.at[c]], irows_v.at[pl.ds(c * CHUNK, CHUNK)],
            isem))
    for cp in copies:
        cp.wait()

    wvecs = [w_v[pl.ds(j * L, L)] for j in range(D // L)]
    bvec = b_v[...]
    liota = lax.iota(jnp.int32, L)

    def group_body(g, carry):
        r0 = g * L
        acc = bvec
        for rr in range(L):
            r = r0 + rr
            s = None
            for j in range(D // L):
                t = (urows_v[r, pl.ds(j * L, L)]
                     * irows_v[r, pl.ds(j * L, L)] * wvecs[j])
                s = t if s is None else s + t
            acc = jnp.where(liota == rr, acc + jnp.sum(s), acc)
        out_v[pl.ds(r0, L)] = acc
        return carry

    lax.fori_loop(0, NGROUP, group_body, 0, unroll=False)

    pltpu.sync_copy(out_v, out_hbm.at[pl.ds(base, BPW)])


def kernel(user_ids, item_ids, user_table, item_table, fc_w, fc_b):
    w = fc_w.reshape(D)
    b = jnp.broadcast_to(fc_b.reshape(1), (L,))
    return _sc_kernel(user_ids, item_ids, user_table, item_table, w, b)
